# baseline (speedup 1.0000x reference)
import functools

import jax
import jax.numpy as jnp
from jax import lax
from jax.experimental import pallas as pl
from jax.experimental.pallas import tpu as pltpu
from jax.experimental.pallas import tpu_sc as plsc

B = 16384
D = 32
V = 1000000
KP1 = 21
NC, NS = 2, 16
NW = NC * NS
PER_W = B // NW
CHUNK = 128
NCHUNK = PER_W // CHUNK
TOTCH = NW * NCHUNK
TBLK = 8192


def _tc_to_rowmajor(table):
    tt = table.T

    def body(x_ref, o_ref):
        r = lax.broadcasted_iota(jnp.int32, (D, D), 0)
        c = lax.broadcasted_iota(jnp.int32, (D, D), 1)
        ident = jnp.where(r == c, 1.0, 0.0).astype(jnp.float32)
        o_ref[...] = lax.dot_general(
            x_ref[...], ident, (((0,), (0,)), ((), ())),
            preferred_element_type=jnp.float32)

    return pl.pallas_call(
        body,
        grid=(pl.cdiv(V, TBLK),),
        in_specs=[pl.BlockSpec((D, TBLK), lambda i: (0, i))],
        out_specs=pl.BlockSpec((TBLK, D), lambda i: (i, 0)),
        out_shape=jax.ShapeDtypeStruct((V, D), jnp.float32),
    )(tt)


def _sc_scores(cen_ch, cidx_ch, inr, outr):
    mesh = plsc.VectorSubcoreMesh(
        core_axis_name="c", subcore_axis_name="s",
        num_cores=NC, num_subcores=NS)

    @functools.partial(
        pl.kernel,
        out_type=jax.ShapeDtypeStruct((TOTCH, KP1, CHUNK), jnp.float32),
        mesh=mesh,
        compiler_params=pltpu.CompilerParams(
            use_tc_tiling_on_sc=False, needs_layout_passes=False),
        scratch_types=[
            pltpu.VMEM((CHUNK,), jnp.int32),
            pltpu.VMEM((KP1, CHUNK), jnp.int32),
            pltpu.VMEM((CHUNK, D), jnp.float32),
            pltpu.VMEM((KP1, CHUNK, D), jnp.float32),
            pltpu.VMEM((KP1, CHUNK), jnp.float32),
            pltpu.SemaphoreType.DMA,
        ],
    )
    def k(cen_hbm, cidx_hbm, inr_hbm, outr_hbm, out_hbm,
          cen_v, cidx_v, crows_v, orows_v, sc_v, sem):
        wid = lax.axis_index("s") * NC + lax.axis_index("c")
        iota = lax.iota(jnp.int32, 16)

        def chunk_body(c, _):
            ch = wid * NCHUNK + c
            pltpu.sync_copy(cen_hbm.at[ch], cen_v)
            pltpu.sync_copy(cidx_hbm.at[ch], cidx_v)
            cps = [pltpu.async_copy(inr_hbm.at[cen_v], crows_v, sem)]
            for j in range(KP1):
                cps.append(pltpu.async_copy(
                    outr_hbm.at[cidx_v.at[j]], orows_v.at[j], sem))
            for cp in cps:
                cp.wait()

            def g_body(g, _):
                r16 = g * 16 + iota
                ccols = [
                    plsc.load_gather(
                        crows_v, [r16, jnp.full((16,), d, jnp.int32)])
                    for d in range(D)
                ]
                for j in range(KP1):
                    jj = jnp.full((16,), j, jnp.int32)
                    s = ccols[0] * plsc.load_gather(
                        orows_v, [jj, r16, jnp.full((16,), 0, jnp.int32)])
                    for d in range(1, D):
                        s = s + ccols[d] * plsc.load_gather(
                            orows_v, [jj, r16, jnp.full((16,), d, jnp.int32)])
                    if j:
                        s = -s
                    sc_v[j, pl.ds(g * 16, 16)] = s
                return 0

            lax.fori_loop(0, CHUNK // 16, g_body, 0)
            pltpu.sync_copy(sc_v, out_hbm.at[ch])
            return 0

        lax.fori_loop(0, NCHUNK, chunk_body, 0)

    return k(cen_ch, cidx_ch, inr, outr)


def _tc_loss(scores):
    x2 = scores.reshape(B * KP1 // 128, 128)

    def body(x_ref, o_ref):
        x = x_ref[...]
        ls = jnp.minimum(x, 0.0) - jnp.log1p(jnp.exp(-jnp.abs(x)))
        o_ref[0, 0] = -jnp.sum(ls) * (1.0 / B)

    out = pl.pallas_call(
        body,
        out_shape=jax.ShapeDtypeStruct((1, 1), jnp.float32),
        out_specs=pl.BlockSpec(memory_space=pltpu.SMEM),
    )(x2)
    return out[0, 0]


def kernel(center, context, negatives, in_embed, out_embed):
    cidx = jnp.concatenate([context[:, None], negatives], axis=1)
    cidx_ch = cidx.reshape(TOTCH, CHUNK, KP1).transpose(0, 2, 1)
    cen_ch = center.reshape(TOTCH, CHUNK)
    inr = _tc_to_rowmajor(in_embed)
    outr = _tc_to_rowmajor(out_embed)
    scores = _sc_scores(cen_ch, cidx_ch, inr, outr)
    return _tc_loss(scores)


# speedup vs baseline: 1.0075x; 1.0075x over previous
import functools

import jax
import jax.numpy as jnp
from jax import lax
from jax.experimental import pallas as pl
from jax.experimental.pallas import tpu as pltpu
from jax.experimental.pallas import tpu_sc as plsc

B = 16384
D = 32
V = 1000000
KP1 = 21
NC, NS = 2, 16
NW = NC * NS
PER_W = B // NW
CHUNK = 128
NCHUNK = PER_W // CHUNK
TOTCH = NW * NCHUNK
TBLK = 8192


def _tc_to_rowmajor(table):
    tt = table.T

    def body(x0, x1, x2, x3, o_ref):
        x = jnp.concatenate([x0[...], x1[...], x2[...], x3[...]], axis=0)
        o_ref[...] = x.T

    return pl.pallas_call(
        body,
        grid=(pl.cdiv(V, TBLK),),
        in_specs=[
            pl.BlockSpec((8, TBLK), functools.partial(
                lambda q, i: (q, i), q)) for q in range(4)
        ],
        out_specs=pl.BlockSpec((TBLK, D), lambda i: (i, 0)),
        out_shape=jax.ShapeDtypeStruct((V, D), jnp.float32),
    )(tt, tt, tt, tt)


def _sc_scores(cen_ch, cidx_ch, inr, outr):
    mesh = plsc.VectorSubcoreMesh(
        core_axis_name="c", subcore_axis_name="s",
        num_cores=NC, num_subcores=NS)

    @functools.partial(
        pl.kernel,
        out_type=jax.ShapeDtypeStruct((TOTCH, KP1, CHUNK), jnp.float32),
        mesh=mesh,
        compiler_params=pltpu.CompilerParams(
            use_tc_tiling_on_sc=False, needs_layout_passes=False),
        scratch_types=[
            pltpu.VMEM((CHUNK,), jnp.int32),
            pltpu.VMEM((KP1, CHUNK), jnp.int32),
            pltpu.VMEM((CHUNK, D), jnp.float32),
            pltpu.VMEM((KP1, CHUNK, D), jnp.float32),
            pltpu.VMEM((KP1, CHUNK), jnp.float32),
            pltpu.SemaphoreType.DMA,
        ],
    )
    def k(cen_hbm, cidx_hbm, inr_hbm, outr_hbm, out_hbm,
          cen_v, cidx_v, crows_v, orows_v, sc_v, sem):
        wid = lax.axis_index("s") * NC + lax.axis_index("c")
        iota = lax.iota(jnp.int32, 16)

        def chunk_body(c, _):
            ch = wid * NCHUNK + c
            pltpu.sync_copy(cen_hbm.at[ch], cen_v)
            pltpu.sync_copy(cidx_hbm.at[ch], cidx_v)
            cps = [pltpu.async_copy(inr_hbm.at[cen_v], crows_v, sem)]
            for j in range(KP1):
                cps.append(pltpu.async_copy(
                    outr_hbm.at[cidx_v.at[j]], orows_v.at[j], sem))
            for cp in cps:
                cp.wait()

            def g_body(g, _):
                r16 = g * 16 + iota
                ccols = [
                    plsc.load_gather(
                        crows_v, [r16, jnp.full((16,), d, jnp.int32)])
                    for d in range(D)
                ]
                for j in range(KP1):
                    jj = jnp.full((16,), j, jnp.int32)
                    s = ccols[0] * plsc.load_gather(
                        orows_v, [jj, r16, jnp.full((16,), 0, jnp.int32)])
                    for d in range(1, D):
                        s = s + ccols[d] * plsc.load_gather(
                            orows_v, [jj, r16, jnp.full((16,), d, jnp.int32)])
                    if j:
                        s = -s
                    sc_v[j, pl.ds(g * 16, 16)] = s
                return 0

            lax.fori_loop(0, CHUNK // 16, g_body, 0)
            pltpu.sync_copy(sc_v, out_hbm.at[ch])
            return 0

        lax.fori_loop(0, NCHUNK, chunk_body, 0)

    return k(cen_ch, cidx_ch, inr, outr)


def _tc_loss(scores):
    x2 = scores.reshape(B * KP1 // 128, 128)

    def body(x_ref, o_ref):
        x = x_ref[...]
        ls = jnp.minimum(x, 0.0) - jnp.log1p(jnp.exp(-jnp.abs(x)))
        o_ref[0, 0] = -jnp.sum(ls) * (1.0 / B)

    out = pl.pallas_call(
        body,
        out_shape=jax.ShapeDtypeStruct((1, 1), jnp.float32),
        out_specs=pl.BlockSpec(memory_space=pltpu.SMEM),
    )(x2)
    return out[0, 0]


def kernel(center, context, negatives, in_embed, out_embed):
    cidx = jnp.concatenate([context[:, None], negatives], axis=1)
    cidx_ch = cidx.reshape(TOTCH, CHUNK, KP1).transpose(0, 2, 1)
    cen_ch = center.reshape(TOTCH, CHUNK)
    inr = _tc_to_rowmajor(in_embed)
    outr = _tc_to_rowmajor(out_embed)
    scores = _sc_scores(cen_ch, cidx_ch, inr, outr)
    return _tc_loss(scores)


# speedup vs baseline: 1.0670x; 1.0590x over previous
import functools

import jax
import jax.numpy as jnp
from jax import lax
from jax.experimental import pallas as pl
from jax.experimental.pallas import tpu as pltpu
from jax.experimental.pallas import tpu_sc as plsc

B = 16384
D = 32
V = 1000000
KP1 = 21
NC, NS = 2, 16
NW = NC * NS
PER_W = B // NW
CHUNK = 128
NCHUNK = PER_W // CHUNK
TOTCH = NW * NCHUNK
TBLK = 32768


def _tc_to_rowmajor(table):
    tt = table.T

    def body(x0, x1, x2, x3, o_ref):
        x = jnp.concatenate([x0[...], x1[...], x2[...], x3[...]], axis=0)
        o_ref[...] = x.T

    return pl.pallas_call(
        body,
        grid=(pl.cdiv(V, TBLK),),
        in_specs=[
            pl.BlockSpec((8, TBLK), functools.partial(
                lambda q, i: (q, i), q)) for q in range(4)
        ],
        out_specs=pl.BlockSpec((TBLK, D), lambda i: (i, 0)),
        out_shape=jax.ShapeDtypeStruct((V, D), jnp.float32),
    )(tt, tt, tt, tt)


def _sc_scores(cen_ch, cidx_ch, inr, outr):
    mesh = plsc.VectorSubcoreMesh(
        core_axis_name="c", subcore_axis_name="s",
        num_cores=NC, num_subcores=NS)

    @functools.partial(
        pl.kernel,
        out_type=jax.ShapeDtypeStruct((TOTCH, KP1, CHUNK), jnp.float32),
        mesh=mesh,
        compiler_params=pltpu.CompilerParams(
            use_tc_tiling_on_sc=False, needs_layout_passes=False),
        scratch_types=[
            pltpu.VMEM((CHUNK,), jnp.int32),
            pltpu.VMEM((KP1, CHUNK), jnp.int32),
            pltpu.VMEM((CHUNK, D), jnp.float32),
            pltpu.VMEM((KP1, CHUNK, D), jnp.float32),
            pltpu.VMEM((KP1, CHUNK), jnp.float32),
            pltpu.SemaphoreType.DMA,
        ],
    )
    def k(cen_hbm, cidx_hbm, inr_hbm, outr_hbm, out_hbm,
          cen_v, cidx_v, crows_v, orows_v, sc_v, sem):
        wid = lax.axis_index("s") * NC + lax.axis_index("c")
        iota = lax.iota(jnp.int32, 16)

        def chunk_body(c, _):
            ch = wid * NCHUNK + c
            pltpu.sync_copy(cen_hbm.at[ch], cen_v)
            pltpu.sync_copy(cidx_hbm.at[ch], cidx_v)
            cps = [pltpu.async_copy(inr_hbm.at[cen_v], crows_v, sem)]
            for j in range(KP1):
                cps.append(pltpu.async_copy(
                    outr_hbm.at[cidx_v.at[j]], orows_v.at[j], sem))
            for cp in cps:
                cp.wait()

            def g_body(g, _):
                r16 = g * 16 + iota
                ccols = [
                    plsc.load_gather(
                        crows_v, [r16, jnp.full((16,), d, jnp.int32)])
                    for d in range(D)
                ]
                for j in range(KP1):
                    jj = jnp.full((16,), j, jnp.int32)
                    s = ccols[0] * plsc.load_gather(
                        orows_v, [jj, r16, jnp.full((16,), 0, jnp.int32)])
                    for d in range(1, D):
                        s = s + ccols[d] * plsc.load_gather(
                            orows_v, [jj, r16, jnp.full((16,), d, jnp.int32)])
                    if j:
                        s = -s
                    sc_v[j, pl.ds(g * 16, 16)] = s
                return 0

            lax.fori_loop(0, CHUNK // 16, g_body, 0)
            pltpu.sync_copy(sc_v, out_hbm.at[ch])
            return 0

        lax.fori_loop(0, NCHUNK, chunk_body, 0)

    return k(cen_ch, cidx_ch, inr, outr)


def _tc_loss(scores):
    x2 = scores.reshape(B * KP1 // 128, 128)

    def body(x_ref, o_ref):
        x = x_ref[...]
        ls = jnp.minimum(x, 0.0) - jnp.log1p(jnp.exp(-jnp.abs(x)))
        o_ref[0, 0] = -jnp.sum(ls) * (1.0 / B)

    out = pl.pallas_call(
        body,
        out_shape=jax.ShapeDtypeStruct((1, 1), jnp.float32),
        out_specs=pl.BlockSpec(memory_space=pltpu.SMEM),
    )(x2)
    return out[0, 0]


def kernel(center, context, negatives, in_embed, out_embed):
    cidx = jnp.concatenate([context[:, None], negatives], axis=1)
    cidx_ch = cidx.reshape(TOTCH, CHUNK, KP1).transpose(0, 2, 1)
    cen_ch = center.reshape(TOTCH, CHUNK)
    inr = _tc_to_rowmajor(in_embed)
    outr = _tc_to_rowmajor(out_embed)
    scores = _sc_scores(cen_ch, cidx_ch, inr, outr)
    return _tc_loss(scores)


# speedup vs baseline: 1.2722x; 1.1924x over previous
import functools

import jax
import jax.numpy as jnp
from jax import lax
from jax.experimental import pallas as pl
from jax.experimental.pallas import tpu as pltpu
from jax.experimental.pallas import tpu_sc as plsc

B = 16384
D = 32
V = 1000000
KP1 = 21
NC, NS = 2, 16
NW = NC * NS
PER_W = B // NW
CHUNK = 64
NCHUNK = PER_W // CHUNK
TOTCH = NW * NCHUNK


def _sc_scores(cen_ch, cidx_ch, inr, outr):
    mesh = plsc.VectorSubcoreMesh(
        core_axis_name="c", subcore_axis_name="s",
        num_cores=NC, num_subcores=NS)

    @functools.partial(
        pl.kernel,
        out_type=jax.ShapeDtypeStruct((TOTCH, KP1, CHUNK), jnp.float32),
        mesh=mesh,
        compiler_params=pltpu.CompilerParams(
            use_tc_tiling_on_sc=False, needs_layout_passes=False),
        scratch_types=[
            pltpu.VMEM((2, CHUNK), jnp.int32),
            pltpu.VMEM((2, KP1, CHUNK), jnp.int32),
            pltpu.VMEM((2, CHUNK, D), jnp.float32),
            pltpu.VMEM((2, KP1, CHUNK, D), jnp.float32),
            pltpu.VMEM((2, KP1, CHUNK), jnp.float32),
            pltpu.SemaphoreType.DMA,
            pltpu.SemaphoreType.DMA,
        ],
    )
    def k(cen_hbm, cidx_hbm, inr_hbm, outr_hbm, out_hbm,
          cen_v, cidx_v, crows_v, orows_v, sc_v, sem0, sem1):
        wid = lax.axis_index("s") * NC + lax.axis_index("c")
        iota = lax.iota(jnp.int32, 16)
        sems = (sem0, sem1)

        def stage_and_fire(c, b):
            ch = wid * NCHUNK + c
            pltpu.sync_copy(cen_hbm.at[ch], cen_v.at[b])
            pltpu.sync_copy(cidx_hbm.at[ch], cidx_v.at[b])
            cps = [pltpu.async_copy(
                inr_hbm.at[cen_v.at[b]], crows_v.at[b], sems[b])]
            for j in range(KP1):
                cps.append(pltpu.async_copy(
                    outr_hbm.at[cidx_v.at[b, j]], orows_v.at[b, j],
                    sems[b]))
            return cps

        def wait_gathers(b):
            pltpu.make_async_copy(
                inr_hbm.at[cen_v.at[b]], crows_v.at[b], sems[b]).wait()
            for j in range(KP1):
                pltpu.make_async_copy(
                    outr_hbm.at[cidx_v.at[b, j]], orows_v.at[b, j],
                    sems[b]).wait()

        def compute(c, b):
            def g_body(g, _):
                r16 = g * 16 + iota
                ccols = [
                    plsc.load_gather(
                        crows_v,
                        [jnp.full((16,), b, jnp.int32), r16,
                         jnp.full((16,), d, jnp.int32)])
                    for d in range(D)
                ]
                bb = jnp.full((16,), b, jnp.int32)
                for j in range(KP1):
                    jj = jnp.full((16,), j, jnp.int32)
                    s = ccols[0] * plsc.load_gather(
                        orows_v, [bb, jj, r16, jnp.full((16,), 0, jnp.int32)])
                    for d in range(1, D):
                        s = s + ccols[d] * plsc.load_gather(
                            orows_v,
                            [bb, jj, r16, jnp.full((16,), d, jnp.int32)])
                    if j:
                        s = -s
                    sc_v[b, j, pl.ds(g * 16, 16)] = s
                return 0

            lax.fori_loop(0, CHUNK // 16, g_body, 0)
            pltpu.sync_copy(sc_v.at[b], out_hbm.at[wid * NCHUNK + c])

        stage_and_fire(0, 0)

        def pair_body(c2, _):
            for b in range(2):
                c = c2 * 2 + b
                wait_gathers(b)

                @pl.when(c + 1 < NCHUNK)
                def _():
                    stage_and_fire(c + 1, 1 - b)

                compute(c, b)
            return 0

        lax.fori_loop(0, NCHUNK // 2, pair_body, 0)

    return k(cen_ch, cidx_ch, inr, outr)


def _tc_loss(scores):
    x2 = scores.reshape(B * KP1 // 128, 128)

    def body(x_ref, o_ref):
        x = x_ref[...]
        ls = jnp.minimum(x, 0.0) - jnp.log1p(jnp.exp(-jnp.abs(x)))
        o_ref[0, 0] = -jnp.sum(ls) * (1.0 / B)

    out = pl.pallas_call(
        body,
        out_shape=jax.ShapeDtypeStruct((1, 1), jnp.float32),
        out_specs=pl.BlockSpec(memory_space=pltpu.SMEM),
    )(x2)
    return out[0, 0]


def kernel(center, context, negatives, in_embed, out_embed):
    cidx = jnp.concatenate([context[:, None], negatives], axis=1)
    cidx_ch = cidx.reshape(TOTCH, CHUNK, KP1).transpose(0, 2, 1)
    cen_ch = center.reshape(TOTCH, CHUNK)
    scores = _sc_scores(cen_ch, cidx_ch, in_embed, out_embed)
    return _tc_loss(scores)
